# baseline (speedup 1.0000x reference)
import functools
import math

import jax
import jax.numpy as jnp
from jax import lax
from jax.experimental import pallas as pl
from jax.experimental.pallas import tpu as pltpu
from jax.experimental.pallas import tpu_sc as plsc

D_MODEL = 1024
SCALE = math.sqrt(D_MODEL)
L = 16

NUM_CORES = 2
NUM_SUBCORES = 16
NW = NUM_CORES * NUM_SUBCORES

B_TOTAL = 4 * 8192
B_PER_W = B_TOTAL // NW
CHUNK = 32
N_CHUNKS = B_PER_W // CHUNK


def _make_kernel():
    mesh = plsc.VectorSubcoreMesh(
        core_axis_name="c", subcore_axis_name="s",
        num_cores=NUM_CORES, num_subcores=NUM_SUBCORES)

    @functools.partial(
        pl.kernel,
        out_type=jax.ShapeDtypeStruct((B_TOTAL, D_MODEL), jnp.float32),
        mesh=mesh,
        scratch_types=[
            pltpu.VMEM((B_PER_W,), jnp.int32),
            pltpu.VMEM((CHUNK, D_MODEL), jnp.float32),
            pltpu.SemaphoreType.DMA,
        ],
    )
    def emb(x_hbm, table_hbm, out_hbm, idx_v, rows_v, gsem):
        wid = lax.axis_index("s") * NUM_CORES + lax.axis_index("c")
        base = wid * B_PER_W
        pltpu.sync_copy(x_hbm.at[pl.ds(base, B_PER_W)], idx_v)

        def chunk_body(c, carry):
            idxs = idx_v.at[pl.ds(c * CHUNK, CHUNK)]
            pltpu.async_copy(table_hbm.at[idxs], rows_v, gsem).wait()

            def row_body(r, carry2):
                def col_body(j, carry3):
                    v = rows_v[r, pl.ds(j * L, L)]
                    rows_v[r, pl.ds(j * L, L)] = v * SCALE
                    return carry3
                return lax.fori_loop(0, D_MODEL // L, col_body, carry2)
            lax.fori_loop(0, CHUNK, row_body, 0)

            pltpu.sync_copy(rows_v, out_hbm.at[pl.ds(base + c * CHUNK, CHUNK)])
            return carry
        lax.fori_loop(0, N_CHUNKS, chunk_body, 0)

    return emb


_emb = _make_kernel()


def kernel(x, table):
    x_flat = x.reshape(-1).astype(jnp.int32)
    out = _emb(x_flat, table)
    return out.reshape(x.shape + (D_MODEL,))


# speedup vs baseline: 3.6685x; 3.6685x over previous
import functools
import math

import jax
import jax.numpy as jnp
from jax import lax
from jax.experimental import pallas as pl
from jax.experimental.pallas import tpu as pltpu
from jax.experimental.pallas import tpu_sc as plsc

D_MODEL = 1024
SCALE = math.sqrt(D_MODEL)
L = 16

NUM_CORES = 2
NUM_SUBCORES = 16
NW = NUM_CORES * NUM_SUBCORES

B_TOTAL = 4 * 8192
B_PER_W = B_TOTAL // NW
CHUNK = 32
N_CHUNKS = B_PER_W // CHUNK


def _make_kernel():
    mesh = plsc.VectorSubcoreMesh(
        core_axis_name="c", subcore_axis_name="s",
        num_cores=NUM_CORES, num_subcores=NUM_SUBCORES)

    @functools.partial(
        pl.kernel,
        out_type=jax.ShapeDtypeStruct((B_TOTAL, D_MODEL), jnp.float32),
        mesh=mesh,
        scratch_types=[
            pltpu.VMEM((B_PER_W,), jnp.int32),
            pltpu.VMEM((2, CHUNK, D_MODEL), jnp.float32),
            pltpu.SemaphoreType.DMA,
            pltpu.SemaphoreType.DMA,
            pltpu.SemaphoreType.DMA,
            pltpu.SemaphoreType.DMA,
        ],
    )
    def emb(x_hbm, table_hbm, out_hbm, idx_v, rows_v, gsem0, gsem1,
            ssem0, ssem1):
        wid = lax.axis_index("s") * NUM_CORES + lax.axis_index("c")
        base = wid * B_PER_W
        pltpu.sync_copy(x_hbm.at[pl.ds(base, B_PER_W)], idx_v)

        gsems = (gsem0, gsem1)
        ssems = (ssem0, ssem1)

        def gather(c, p):
            idxs = idx_v.at[pl.ds(c * CHUNK, CHUNK)]
            return pltpu.make_async_copy(table_hbm.at[idxs], rows_v.at[p],
                                         gsems[p])

        def store(c, p):
            return pltpu.make_async_copy(
                rows_v.at[p], out_hbm.at[pl.ds(base + c * CHUNK, CHUNK)],
                ssems[p])

        gather(0, 0).start()

        def pair_body(co, carry):
            for p in range(2):
                c = co * 2 + p
                other = 1 - p

                @pl.when(c >= 1)
                def _():
                    store(c - 1, other).wait()

                @pl.when(c + 1 < N_CHUNKS)
                def _():
                    gather(c + 1, other).start()

                gather(c, p).wait()

                def row_body(r, carry2):
                    for j in range(D_MODEL // L):
                        v = rows_v[p, r, pl.ds(j * L, L)]
                        rows_v[p, r, pl.ds(j * L, L)] = v * SCALE
                    return carry2
                lax.fori_loop(0, CHUNK, row_body, 0)

                store(c, p).start()
            return carry
        lax.fori_loop(0, N_CHUNKS // 2, pair_body, 0)

        store(N_CHUNKS - 1, 1).wait()

    return emb


_emb = _make_kernel()


def kernel(x, table):
    x_flat = x.reshape(-1).astype(jnp.int32)
    out = _emb(x_flat, table)
    return out.reshape(x.shape + (D_MODEL,))


# speedup vs baseline: 4.0798x; 1.1121x over previous
import functools
import math

import jax
import jax.numpy as jnp
from jax import lax
from jax.experimental import pallas as pl
from jax.experimental.pallas import tpu as pltpu
from jax.experimental.pallas import tpu_sc as plsc

D_MODEL = 1024
SCALE = math.sqrt(D_MODEL)
L = 16

NUM_CORES = 2
NUM_SUBCORES = 16
NW = NUM_CORES * NUM_SUBCORES

B_TOTAL = 4 * 8192
B_PER_W = B_TOTAL // NW
CHUNK = 16
N_CHUNKS = B_PER_W // CHUNK
NBUF = 4


def _make_kernel():
    mesh = plsc.VectorSubcoreMesh(
        core_axis_name="c", subcore_axis_name="s",
        num_cores=NUM_CORES, num_subcores=NUM_SUBCORES)

    @functools.partial(
        pl.kernel,
        out_type=jax.ShapeDtypeStruct((B_TOTAL, D_MODEL), jnp.float32),
        mesh=mesh,
        scratch_types=[
            pltpu.VMEM((B_PER_W,), jnp.int32),
            pltpu.VMEM((NBUF, CHUNK, D_MODEL), jnp.float32),
        ] + [pltpu.SemaphoreType.DMA] * (2 * NBUF),
    )
    def emb(x_hbm, table_hbm, out_hbm, idx_v, rows_v, *sems):
        gsems = sems[:NBUF]
        ssems = sems[NBUF:]
        wid = lax.axis_index("s") * NUM_CORES + lax.axis_index("c")
        base = wid * B_PER_W
        pltpu.sync_copy(x_hbm.at[pl.ds(base, B_PER_W)], idx_v)

        def gather(c, p):
            idxs = idx_v.at[pl.ds(c * CHUNK, CHUNK)]
            return pltpu.make_async_copy(table_hbm.at[idxs], rows_v.at[p],
                                         gsems[p])

        def store(c, p):
            return pltpu.make_async_copy(
                rows_v.at[p], out_hbm.at[pl.ds(base + c * CHUNK, CHUNK)],
                ssems[p])

        for c in range(NBUF - 1):
            gather(c, c).start()

        def ring_body(co, carry):
            for p in range(NBUF):
                c = co * NBUF + p
                gather(c, p).wait()

                def row_body(r, carry2):
                    for j in range(D_MODEL // L):
                        v = rows_v[p, r, pl.ds(j * L, L)]
                        rows_v[p, r, pl.ds(j * L, L)] = v * SCALE
                    return carry2
                lax.fori_loop(0, CHUNK, row_body, 0)

                store(c, p).start()

                nxt = c + NBUF - 1
                pnxt = (p + NBUF - 1) % NBUF

                @pl.when(nxt < N_CHUNKS)
                def _():
                    @pl.when(nxt >= NBUF)
                    def _():
                        store(nxt - NBUF, pnxt).wait()
                    gather(nxt, pnxt).start()
            return carry
        lax.fori_loop(0, N_CHUNKS // NBUF, ring_body, 0)

        for c in range(N_CHUNKS - NBUF, N_CHUNKS):
            store(c, c % NBUF).wait()

    return emb


_emb = _make_kernel()


def kernel(x, table):
    x_flat = x.reshape(-1).astype(jnp.int32)
    out = _emb(x_flat, table)
    return out.reshape(x.shape + (D_MODEL,))
